# baseline (speedup 1.0000x reference)
import jax
import jax.numpy as jnp
from jax.experimental import pallas as pl
from jax.experimental.pallas import tpu as pltpu

_CHUNK = 2000
_NBUF = 10


def _body(x_hbm, w1t_ref, b1_ref, w2t_ref, b2_ref, o_hbm,
          x_buf, o_buf, in_sem, out_sem):
    n = x_hbm.shape[0]
    nchunks = n // _CHUNK
    w1t = w1t_ref[...]
    b1 = b1_ref[...]
    w2t = w2t_ref[...]
    b2 = b2_ref[...]

    def in_copy(i):
        slot = i % _NBUF
        return pltpu.make_async_copy(
            x_hbm.at[pl.ds(i * _CHUNK, _CHUNK), :],
            x_buf.at[slot], in_sem.at[slot])

    def out_copy(i):
        slot = i % _NBUF
        return pltpu.make_async_copy(
            o_buf.at[slot],
            o_hbm.at[pl.ds(i * _CHUNK, _CHUNK), :], out_sem.at[slot])

    for k in range(min(_NBUF, nchunks)):
        in_copy(k).start()

    for i in range(nchunks):
        slot = i % _NBUF
        in_copy(i).wait()
        xb = x_buf[slot].astype(jnp.bfloat16)
        h = jnp.dot(xb, w1t, preferred_element_type=jnp.float32)
        h = jnp.maximum(h + b1, 0.0).astype(jnp.bfloat16)
        if i >= _NBUF:
            out_copy(i - _NBUF).wait()
        o_buf[slot] = jnp.dot(h, w2t, preferred_element_type=jnp.float32) + b2
        out_copy(i).start()
        if i + _NBUF < nchunks:
            in_copy(i + _NBUF).start()

    for i in range(max(nchunks - _NBUF, 0), nchunks):
        out_copy(i).wait()


def kernel(x, W1, b1, W2, b2):
    n, d = x.shape
    h_dim = W1.shape[0]
    out_dim = W2.shape[0]
    w1t = W1.T.astype(jnp.bfloat16)
    w2t = W2.T.astype(jnp.bfloat16)
    b1r = b1.reshape(1, h_dim)
    b2r = b2.reshape(1, out_dim)
    anyspec = pl.BlockSpec(memory_space=pltpu.MemorySpace.HBM)
    vmemspec = pl.BlockSpec(memory_space=pltpu.MemorySpace.VMEM)
    return pl.pallas_call(
        _body,
        in_specs=[anyspec, vmemspec, vmemspec, vmemspec, vmemspec],
        out_specs=anyspec,
        out_shape=jax.ShapeDtypeStruct((n, out_dim), jnp.float32),
        scratch_shapes=[
            pltpu.VMEM((_NBUF, _CHUNK, d), jnp.float32),
            pltpu.VMEM((_NBUF, _CHUNK, out_dim), jnp.float32),
            pltpu.SemaphoreType.DMA((_NBUF,)),
            pltpu.SemaphoreType.DMA((_NBUF,)),
        ],
    )(x, w1t, b1r, w2t, b2r)


# speedup vs baseline: 1.6097x; 1.6097x over previous
import jax
import jax.numpy as jnp
from jax.experimental import pallas as pl
from jax.experimental.pallas import tpu as pltpu

_BLOCK_ROWS = 25000


def _mlp_body(x_ref, w1t_ref, b1_ref, w2t_ref, b2_ref, o_ref):
    xb = x_ref[...].astype(jnp.bfloat16)
    h = jnp.dot(xb, w1t_ref[...], preferred_element_type=jnp.float32)
    h = jnp.maximum(h + b1_ref[...], 0.0).astype(jnp.bfloat16)
    o = jnp.dot(h, w2t_ref[...], preferred_element_type=jnp.float32)
    o_ref[...] = o + b2_ref[...]


def kernel(x, W1, b1, W2, b2):
    n, d = x.shape
    h_dim = W1.shape[0]
    out_dim = W2.shape[0]
    w1t = W1.T.astype(jnp.bfloat16)
    w2t = W2.T.astype(jnp.bfloat16)
    b1r = b1.reshape(1, h_dim)
    b2r = b2.reshape(1, out_dim)
    grid = (n // _BLOCK_ROWS,)
    return pl.pallas_call(
        _mlp_body,
        grid=grid,
        in_specs=[
            pl.BlockSpec((_BLOCK_ROWS, d), lambda i: (i, 0)),
            pl.BlockSpec((d, h_dim), lambda i: (0, 0)),
            pl.BlockSpec((1, h_dim), lambda i: (0, 0)),
            pl.BlockSpec((h_dim, out_dim), lambda i: (0, 0)),
            pl.BlockSpec((1, out_dim), lambda i: (0, 0)),
        ],
        out_specs=pl.BlockSpec((_BLOCK_ROWS, out_dim), lambda i: (i, 0)),
        out_shape=jax.ShapeDtypeStruct((n, out_dim), jnp.float32),
        compiler_params=pltpu.CompilerParams(
            dimension_semantics=("parallel",),
            vmem_limit_bytes=120_000_000,
        ),
    )(x, w1t, b1r, w2t, b2r)


# speedup vs baseline: 1.8272x; 1.1351x over previous
import jax
import jax.numpy as jnp
from jax.experimental import pallas as pl
from jax.experimental.pallas import tpu as pltpu

_BLOCK_ROWS = 20000


def _mlp_body(x_ref, w1t_ref, b1_ref, w2t_ref, b2_ref, o_ref):
    h = jnp.dot(x_ref[...], w1t_ref[...], preferred_element_type=jnp.float32)
    h = jnp.maximum(h + b1_ref[...], 0.0)
    o = jnp.dot(h, w2t_ref[...], preferred_element_type=jnp.float32)
    o_ref[...] = o + b2_ref[...]


def kernel(x, W1, b1, W2, b2):
    n, d = x.shape
    h_dim = W1.shape[0]
    out_dim = W2.shape[0]
    w1t = W1.T
    w2t = W2.T
    b1r = b1.reshape(1, h_dim)
    b2r = b2.reshape(1, out_dim)
    grid = (n // _BLOCK_ROWS,)
    return pl.pallas_call(
        _mlp_body,
        grid=grid,
        in_specs=[
            pl.BlockSpec((_BLOCK_ROWS, d), lambda i: (i, 0)),
            pl.BlockSpec((d, h_dim), lambda i: (0, 0)),
            pl.BlockSpec((1, h_dim), lambda i: (0, 0)),
            pl.BlockSpec((h_dim, out_dim), lambda i: (0, 0)),
            pl.BlockSpec((1, out_dim), lambda i: (0, 0)),
        ],
        out_specs=pl.BlockSpec((_BLOCK_ROWS, out_dim), lambda i: (i, 0)),
        out_shape=jax.ShapeDtypeStruct((n, out_dim), jnp.float32),
        compiler_params=pltpu.CompilerParams(
            dimension_semantics=("parallel",),
        ),
    )(x, w1t, b1r, w2t, b2r)
